# Initial kernel scaffold; baseline (speedup 1.0000x reference)
import jax
import jax.numpy as jnp
from jax.experimental import pallas as pl


def kernel(x, edge_index, edge_type, edge_weight, rel_emb, W_proj, b_proj, W1, att_src1, att_dst1, b1, W2, att_src2, att_dst2, b2, g1, be1, g2, be2, W_out, b_out):
    raise NotImplementedError("write your pallas kernel here")


# speedup vs baseline: 1.0001x; 1.0001x over previous
import jax
import jax.numpy as jnp
from jax.experimental import pallas as pl

N = 10000
E = 320000
IN_DIM = 128
CODE_DIM = 96
HID = 256
HEADS = 4
OUT_CH = 64
NUM_REL = 5


def _proj_body(x_ref, wt_ref, b_ref, h_ref):
    x = x_ref[...]
    x_code = x[:, :CODE_DIM]
    x_num = x[:, CODE_DIM:]
    m = jnp.mean(x_num, axis=0, keepdims=True)
    v = jnp.sum((x_num - m) ** 2, axis=0, keepdims=True) / (N - 1)
    s = jnp.clip(jnp.sqrt(v), 1e-6, None)
    xn = (x_num - m) / s
    nrm = jnp.sqrt(jnp.sum(x_code * x_code, axis=1, keepdims=True))
    xc = x_code / jnp.clip(nrm, 1e-12, None)
    xcat = jnp.concatenate([xn, xc], axis=-1)
    h = jnp.dot(xcat, wt_ref[...], preferred_element_type=jnp.float32)
    h_ref[...] = jnp.maximum(h + b_ref[...], 0.0)


def _gat_conv(h, src, dst, W, a_src, a_dst, b):
    n = h.shape[0]
    xp = (h @ W.T).reshape(n, HEADS, OUT_CH)
    alpha_src = (xp * a_src[None]).sum(-1)
    alpha_dst = (xp * a_dst[None]).sum(-1)
    alpha = jax.nn.leaky_relu(alpha_src[src] + alpha_dst[dst], negative_slope=0.2)
    amax = jax.ops.segment_max(alpha, dst, num_segments=n)
    amax = jnp.where(jnp.isfinite(amax), amax, 0.0)
    ex = jnp.exp(alpha - amax[dst])
    den = jax.ops.segment_sum(ex, dst, num_segments=n)
    coef = ex / (den[dst] + 1e-16)
    out = jax.ops.segment_sum(xp[src] * coef[:, :, None], dst, num_segments=n)
    return out.reshape(n, HEADS * OUT_CH) + b


def _layer_norm(x, g, b):
    m = x.mean(axis=-1, keepdims=True)
    v = x.var(axis=-1, keepdims=True)
    return (x - m) / jnp.sqrt(v + 1e-5) * g + b


def kernel(x, edge_index, edge_type, edge_weight, rel_emb, W_proj, b_proj, W1, att_src1, att_dst1, b1, W2, att_src2, att_dst2, b2, g1, be1, g2, be2, W_out, b_out):
    h = pl.pallas_call(
        _proj_body,
        out_shape=jax.ShapeDtypeStruct((N, HID), jnp.float32),
    )(x, W_proj.T, b_proj[None, :])

    rel = rel_emb[edge_type] * edge_weight[:, None]
    msg = h[edge_index[0]] + rel
    h = h.at[edge_index[1]].add(msg)
    ar = jnp.arange(h.shape[0], dtype=edge_index.dtype)
    src = jnp.concatenate([edge_index[0], ar])
    dst = jnp.concatenate([edge_index[1], ar])
    h1 = _gat_conv(h, src, dst, W1, att_src1, att_dst1, b1)
    h = _layer_norm(h1 + h, g1, be1)
    h2 = _gat_conv(h, src, dst, W2, att_src2, att_dst2, b2)
    h = _layer_norm(h2 + h, g2, be2)
    score = (h @ W_out.T + b_out).squeeze(-1)
    return score


# speedup vs baseline: 13.4139x; 13.4127x over previous
import functools

import jax
import jax.numpy as jnp
from jax import lax
from jax.experimental import pallas as pl
from jax.experimental.pallas import tpu as pltpu
from jax.experimental.pallas import tpu_sc as plsc

N = 10000
E = 320000
IN_DIM = 128
CODE_DIM = 96
HID = 256
HEADS = 4
OUT_CH = 64
NUM_REL = 5

HALF = 128
NC = 2
NS = 16
EPT = E // NS
CHUNK = 80
NCHUNK = EPT // CHUNK
WB = 624
WB_EX = N - NS * WB
ZR = 16
ND4 = 640
F32 = jnp.float32
I32 = jnp.int32

_mesh = plsc.VectorSubcoreMesh(core_axis_name="c", subcore_axis_name="s")


def _zero_rows(ref, nrows, ncols):
    zv = jnp.zeros((16,), F32)
    for i in range(nrows):
        for j in range(ncols // 16):
            ref[i, pl.ds(j * 16, 16)] = zv


def _proj_body(x_ref, wt_ref, b_ref, o_ref):
    x = x_ref[...]
    x_code = x[:, :CODE_DIM]
    x_num = x[:, CODE_DIM:]
    m = jnp.mean(x_num, axis=0, keepdims=True)
    v = jnp.sum((x_num - m) ** 2, axis=0, keepdims=True) / (N - 1)
    s = jnp.clip(jnp.sqrt(v), 1e-6, None)
    xn = (x_num - m) / s
    nrm = jnp.sqrt(jnp.sum(x_code * x_code, axis=1, keepdims=True))
    xc = x_code / jnp.clip(nrm, 1e-12, None)
    xcat = jnp.concatenate([xn, xc], axis=-1)
    h = jnp.dot(xcat, wt_ref[...], preferred_element_type=F32)
    h = jnp.maximum(h + b_ref[...], 0.0)
    o_ref[0:N, :] = h[:, :HALF]
    o_ref[N:2 * N, :] = h[:, HALF:]


def _s1_body(h0cat, srce, dste, typee, ewe,
             accH_out, accR_out,
             accS, accRS,
             isrc, idst, idq,
             rows, wrow, typ, ewv, idp,
             zrow,
             gsem, ssem, rsem):
    cid = lax.axis_index("c")
    sid = lax.axis_index("s")
    ebase = sid * EPT
    gbase = cid * N
    iota = jnp.arange(16, dtype=I32)

    _zero_rows(zrow, ZR, HALF)
    for k in range(WB // ZR):
        pltpu.sync_copy(zrow, accS.at[pl.ds(sid * WB + k * ZR, ZR)])
    for k in range(ND4 // NS // ZR):
        pltpu.sync_copy(zrow, accRS.at[pl.ds(sid * (ND4 // NS) + k * ZR, ZR)])
    pltpu.sync_copy(zrow.at[pl.ds(0, 8)],
                    accRS.at[pl.ds(sid * (ND4 // NS) + 2 * ZR, 8)])

    @pl.when(sid == NS - 1)
    def _():
        pltpu.sync_copy(zrow, accS.at[pl.ds(NS * WB, WB_EX)])

    plsc.subcore_barrier()

    def gcopy():
        return pltpu.make_async_copy(h0cat.at[isrc], rows, gsem)

    def scopy():
        return pltpu.make_async_copy(rows, accS.at[idst], ssem)

    def rcopy():
        return pltpu.make_async_copy(wrow, accRS.at[idq], rsem)

    def loop(g, carry):
        off = ebase + g * CHUNK

        @pl.when(g > 0)
        def _():
            scopy().wait()

            @pl.when(cid == 0)
            def _():
                rcopy().wait()

        pltpu.sync_copy(srce.at[pl.ds(off, CHUNK)], isrc)
        for q in range(CHUNK // 16):
            isrc[pl.ds(q * 16, 16)] = isrc[pl.ds(q * 16, 16)] + gbase
        pltpu.sync_copy(dste.at[pl.ds(off, CHUNK)], idst)
        pltpu.sync_copy(dste.at[pl.ds(off, CHUNK)], idp.at[pl.ds(0, CHUNK)])
        for q in range(CHUNK // 16):
            idq[pl.ds(q * 16, 16)] = lax.shift_right_logical(
                idst[pl.ds(q * 16, 16)], 4)
        pltpu.sync_copy(typee.at[pl.ds(off, CHUNK)], typ.at[pl.ds(0, CHUNK)])
        pltpu.sync_copy(ewe.at[pl.ds(off, CHUNK)], ewv.at[pl.ds(0, CHUNK)])
        gcopy().start()

        @pl.when(cid == 0)
        def _():
            def ebody(e, carry2):
                tv = typ[pl.ds(e, 16)]
                wv = ewv[pl.ds(e, 16)]
                dv = idp[pl.ds(e, 16)]
                tgt = 8 * (dv[0] & 15) + tv[0]
                ws = jnp.full((16,), wv[0], F32)
                for j in range(8):
                    wrow[e, pl.ds(j * 16, 16)] = jnp.where(
                        iota == tgt - j * 16, ws, 0.0)
                return carry2

            lax.fori_loop(0, CHUNK, ebody, 0)

        gcopy().wait()
        pltpu.async_copy(rows, accS.at[idst], ssem, add=True)

        @pl.when(cid == 0)
        def _():
            pltpu.async_copy(wrow, accRS.at[idq], rsem, add=True)

        return carry

    lax.fori_loop(0, NCHUNK, loop, 0)
    scopy().wait()

    @pl.when(cid == 0)
    def _():
        rcopy().wait()

    plsc.subcore_barrier()

    sl = pl.ds(sid * WB, WB)
    pltpu.sync_copy(accS.at[sl], accH_out.at[cid, sl])

    @pl.when(cid == 0)
    def _():
        sq = pl.ds(sid * (ND4 // NS), ND4 // NS)
        pltpu.sync_copy(accRS.at[sq], accR_out.at[sq])

    @pl.when(sid == NS - 1)
    def _():
        sx = pl.ds(NS * WB, WB_EX)
        pltpu.sync_copy(accS.at[sx], accH_out.at[cid, sx])


_s1_call = functools.partial(
    pl.kernel,
    _s1_body,
    out_type=[
        jax.ShapeDtypeStruct((NC, N, HALF), F32),
        jax.ShapeDtypeStruct((ND4, HALF), F32),
    ],
    mesh=_mesh,
    scratch_types=[
        pltpu.VMEM_SHARED((N, HALF), F32),
        pltpu.VMEM_SHARED((ND4, HALF), F32),
        pltpu.VMEM((CHUNK,), I32), pltpu.VMEM((CHUNK,), I32),
        pltpu.VMEM((CHUNK,), I32),
        pltpu.VMEM((CHUNK, HALF), F32), pltpu.VMEM((CHUNK, HALF), F32),
        pltpu.VMEM((CHUNK + 16,), I32), pltpu.VMEM((CHUNK + 16,), F32),
        pltpu.VMEM((CHUNK + 16,), I32),
        pltpu.VMEM((ZR, HALF), F32),
        pltpu.SemaphoreType.DMA, pltpu.SemaphoreType.DMA,
        pltpu.SemaphoreType.DMA,
    ],
)


def _mix_half_body(h0c_ref, accHc_ref, accR_ref, relc_ref, o_ref):
    rmat = jnp.dot(accR_ref[0:N, 0:NUM_REL], relc_ref[...],
                   preferred_element_type=F32)
    o_ref[...] = h0c_ref[...] + accHc_ref[...] + rmat


def _xp_half_body(ha_ref, hb_ref, wt1_ref, wt2_ref, o_ref):
    o_ref[...] = (jnp.dot(ha_ref[...], wt1_ref[...], preferred_element_type=F32)
                  + jnp.dot(hb_ref[...], wt2_ref[...], preferred_element_type=F32))


def _stats_body(xp2_ref, wb_ref, ta_ref, td_ref, mx_ref):
    t = jnp.dot(xp2_ref[...], wb_ref[...], preferred_element_type=F32)
    mxs = []
    for h in range(HEADS):
        c = h // 2
        a = t[c * N:(c + 1) * N, h]
        d = t[c * N:(c + 1) * N, 4 + h]
        ta_ref[h, :] = a
        td_ref[h, :] = d
        mxs.append(jnp.max(a))
    lane = jnp.arange(16, dtype=I32)
    e0 = (lane == 0).astype(F32)
    e1 = (lane == 1).astype(F32)
    mx_ref[...] = jnp.concatenate(
        [mxs[0] * e0 + mxs[1] * e1, mxs[2] * e0 + mxs[3] * e1], axis=0)


def _spread_body(t_ref, o_ref):
    z = jnp.zeros((N, 126), F32)
    o_ref[0:N, :] = jnp.concatenate(
        [t_ref[0, :][:, None], t_ref[1, :][:, None], z], axis=1)
    o_ref[N:2 * N, :] = jnp.concatenate(
        [t_ref[2, :][:, None], t_ref[3, :][:, None], z], axis=1)


def _s2_body(xp2, tsrc, tdst, mxf, srce, dste,
             accG_out, accD_out,
             accS, accD,
             isrc, idst, idstg, idq,
             rows, sa, sd, outd, idp, mxb, zrow,
             gsem, ssem, dsem):
    cid = lax.axis_index("c")
    sid = lax.axis_index("s")
    ebase = sid * EPT
    gbase = cid * N
    iota = jnp.arange(16, dtype=I32)

    pltpu.sync_copy(mxf.at[pl.ds(cid * 16, 16)], mxb)

    _zero_rows(zrow, ZR, HALF)
    for k in range(WB // ZR):
        pltpu.sync_copy(zrow, accS.at[pl.ds(sid * WB + k * ZR, ZR)])
    for k in range(ND4 // NS // ZR):
        pltpu.sync_copy(zrow, accD.at[pl.ds(sid * (ND4 // NS) + k * ZR, ZR)])
    pltpu.sync_copy(zrow.at[pl.ds(0, 8)],
                    accD.at[pl.ds(sid * (ND4 // NS) + 2 * ZR, 8)])

    @pl.when(sid == NS - 1)
    def _():
        pltpu.sync_copy(zrow, accS.at[pl.ds(NS * WB, WB_EX)])

    plsc.subcore_barrier()

    def gxcopy():
        return pltpu.make_async_copy(xp2.at[isrc], rows, gsem)

    def gacopy():
        return pltpu.make_async_copy(tsrc.at[isrc], sa, gsem)

    def gdcopy():
        return pltpu.make_async_copy(tdst.at[idstg], sd, gsem)

    def scopy():
        return pltpu.make_async_copy(rows, accS.at[idst], ssem)

    def dcopy():
        return pltpu.make_async_copy(outd, accD.at[idq], dsem)

    def loop(g, carry):
        off = ebase + g * CHUNK

        @pl.when(g > 0)
        def _():
            scopy().wait()
            dcopy().wait()

        pltpu.sync_copy(srce.at[pl.ds(off, CHUNK)], isrc)
        pltpu.sync_copy(dste.at[pl.ds(off, CHUNK)], idst)
        pltpu.sync_copy(dste.at[pl.ds(off, CHUNK)], idp.at[pl.ds(0, CHUNK)])
        for q in range(CHUNK // 16):
            isrc[pl.ds(q * 16, 16)] = isrc[pl.ds(q * 16, 16)] + gbase
            idstg[pl.ds(q * 16, 16)] = idst[pl.ds(q * 16, 16)] + gbase
            idq[pl.ds(q * 16, 16)] = lax.shift_right_logical(
                idst[pl.ds(q * 16, 16)], 4)
        gxcopy().start()
        gacopy().start()
        gdcopy().start()
        gxcopy().wait()
        gacopy().wait()
        gdcopy().wait()

        vmx = mxb[pl.ds(0, 16)]

        def ebody(e, carry2):
            va = sa[e, pl.ds(0, 16)]
            vd = sd[e, pl.ds(0, 16)]
            dv = idp[pl.ds(e, 16)]
            xr = [rows[e, pl.ds(j * 16, 16)] for j in range(8)]
            s = va + vd
            t = vmx + vd
            ex = jnp.exp(jnp.maximum(s, 0.2 * s) - jnp.maximum(t, 0.2 * t))
            b0 = jnp.full((16,), ex[0], F32)
            b1 = jnp.full((16,), ex[1], F32)
            lane = 8 * (dv[0] & 15)
            for j in range(8):
                outd[e, pl.ds(j * 16, 16)] = (
                    jnp.where(iota == lane - j * 16, b0, 0.0)
                    + jnp.where(iota == lane + 1 - j * 16, b1, 0.0))
            for j in range(4):
                rows[e, pl.ds(j * 16, 16)] = xr[j] * b0
            for j in range(4, 8):
                rows[e, pl.ds(j * 16, 16)] = xr[j] * b1
            return carry2

        lax.fori_loop(0, CHUNK, ebody, 0)
        pltpu.async_copy(rows, accS.at[idst], ssem, add=True)
        pltpu.async_copy(outd, accD.at[idq], dsem, add=True)
        return carry

    lax.fori_loop(0, NCHUNK, loop, 0)
    scopy().wait()
    dcopy().wait()
    plsc.subcore_barrier()

    sl = pl.ds(sid * WB, WB)
    pltpu.sync_copy(accS.at[sl], accG_out.at[cid, sl])
    sq = pl.ds(sid * (ND4 // NS), ND4 // NS)
    pltpu.sync_copy(accD.at[sq], accD_out.at[cid, sq])

    @pl.when(sid == NS - 1)
    def _():
        sx = pl.ds(NS * WB, WB_EX)
        pltpu.sync_copy(accS.at[sx], accG_out.at[cid, sx])


_s2_call = functools.partial(
    pl.kernel,
    _s2_body,
    out_type=[
        jax.ShapeDtypeStruct((NC, N, HALF), F32),
        jax.ShapeDtypeStruct((NC, ND4, HALF), F32),
    ],
    mesh=_mesh,
    scratch_types=[
        pltpu.VMEM_SHARED((N, HALF), F32),
        pltpu.VMEM_SHARED((ND4, HALF), F32),
        pltpu.VMEM((CHUNK,), I32), pltpu.VMEM((CHUNK,), I32),
        pltpu.VMEM((CHUNK,), I32), pltpu.VMEM((CHUNK,), I32),
        pltpu.VMEM((CHUNK, HALF), F32), pltpu.VMEM((CHUNK, HALF), F32),
        pltpu.VMEM((CHUNK, HALF), F32), pltpu.VMEM((CHUNK, HALF), F32),
        pltpu.VMEM((CHUNK + 16,), I32),
        pltpu.VMEM((16,), F32),
        pltpu.VMEM((ZR, HALF), F32),
        pltpu.SemaphoreType.DMA, pltpu.SemaphoreType.DMA,
        pltpu.SemaphoreType.DMA,
    ],
)


def _gat_half_body(accG_ref, accD_ref, xp2c_ref, ta2_ref, td2_ref, mxc_ref,
                   bc_ref, hprevc_ref, o_ref):
    mxv = mxc_ref[...]
    for h01 in range(2):
        feats = accG_ref[:, OUT_CH * h01:OUT_CH * (h01 + 1)]
        den = accD_ref[0:N, h01]
        s = ta2_ref[h01, :] + td2_ref[h01, :]
        t = mxv[h01] + td2_ref[h01, :]
        exs = jnp.exp(jnp.maximum(s, 0.2 * s) - jnp.maximum(t, 0.2 * t))
        xph = xp2c_ref[:, OUT_CH * h01:OUT_CH * (h01 + 1)]
        o = (feats + exs[:, None] * xph) / (den + exs + 1e-16)[:, None]
        o_ref[:, OUT_CH * h01:OUT_CH * (h01 + 1)] = (
            o + bc_ref[:, OUT_CH * h01:OUT_CH * (h01 + 1)]
            + hprevc_ref[:, OUT_CH * h01:OUT_CH * (h01 + 1)])


def _lnstat_body(r0_ref, r1_ref, o_ref):
    r0 = r0_ref[...]
    r1 = r1_ref[...]
    m = (jnp.sum(r0, axis=1, keepdims=True)
         + jnp.sum(r1, axis=1, keepdims=True)) / HID
    m2 = (jnp.sum(r0 * r0, axis=1, keepdims=True)
          + jnp.sum(r1 * r1, axis=1, keepdims=True)) / HID
    v = m2 - m * m
    inv = 1.0 / jnp.sqrt(v + 1e-5)
    o_ref[...] = jnp.concatenate([m, inv, jnp.zeros((N, 6), F32)], axis=1)


def _lnapply_body(rc_ref, st_ref, gc_ref, bec_ref, o_ref):
    m = st_ref[:, 0:1]
    inv = st_ref[:, 1:2]
    o_ref[...] = (rc_ref[...] - m) * inv * gc_ref[...] + bec_ref[...]


def _lnfinal_body(r0_ref, r1_ref, st_ref, g_ref, be_ref, wout_ref, bout_ref,
                  o_ref):
    m = st_ref[:, 0:1]
    inv = st_ref[:, 1:2]
    h0 = (r0_ref[...] - m) * inv * g_ref[:, 0:HALF] + be_ref[:, 0:HALF]
    h1 = (r1_ref[...] - m) * inv * g_ref[:, HALF:] + be_ref[:, HALF:]
    score = (jnp.dot(h0, wout_ref[0:HALF, :], preferred_element_type=F32)
             + jnp.dot(h1, wout_ref[HALF:, :], preferred_element_type=F32))
    o_ref[...] = score[:, 0] + bout_ref[0, 0]


def kernel(x, edge_index, edge_type, edge_weight, rel_emb, W_proj, b_proj,
           W1, att_src1, att_dst1, b1, W2, att_src2, att_dst2, b2,
           g1, be1, g2, be2, W_out, b_out):
    src = edge_index[0]
    dst = edge_index[1]
    etype = edge_type.astype(I32)

    h0cat = pl.pallas_call(
        _proj_body,
        out_shape=jax.ShapeDtypeStruct((2 * N, HALF), F32),
    )(x, W_proj.T, b_proj[None, :])

    accH, accRp = _s1_call()(h0cat, src, dst, etype, edge_weight)
    accR = accRp.reshape(16 * ND4, 8)

    def mk_half(body, *args):
        return pl.pallas_call(
            body, out_shape=jax.ShapeDtypeStruct((N, HALF), F32))(*args)

    h1h = [mk_half(_mix_half_body, h0cat[c * N:(c + 1) * N], accH[c], accR,
                   rel_emb[:, c * HALF:(c + 1) * HALF]) for c in range(NC)]

    def gat_layer(hh, W, a_src, a_dst):
        wt = W.T
        xph = [mk_half(_xp_half_body, hh[0], hh[1],
                       wt[0:HALF, c * HALF:(c + 1) * HALF],
                       wt[HALF:, c * HALF:(c + 1) * HALF]) for c in range(NC)]
        xp2 = jnp.concatenate(xph, axis=0)
        wb = jnp.zeros((HALF, 8), F32)
        for h in range(HEADS):
            rs = (h % 2) * OUT_CH
            wb = wb.at[rs:rs + OUT_CH, h].set(a_src[h])
            wb = wb.at[rs:rs + OUT_CH, 4 + h].set(a_dst[h])
        ta, td, mxf = pl.pallas_call(
            _stats_body,
            out_shape=[
                jax.ShapeDtypeStruct((HEADS, N), F32),
                jax.ShapeDtypeStruct((HEADS, N), F32),
                jax.ShapeDtypeStruct((32,), F32),
            ],
        )(xp2, wb)
        tsrc = pl.pallas_call(
            _spread_body,
            out_shape=jax.ShapeDtypeStruct((2 * N, HALF), F32),
        )(ta)
        tdst = pl.pallas_call(
            _spread_body,
            out_shape=jax.ShapeDtypeStruct((2 * N, HALF), F32),
        )(td)
        accG, accDp = _s2_call()(xp2, tsrc, tdst, mxf, src, dst)
        accD = accDp.reshape(NC, 16 * ND4, 8)
        return xp2, ta, td, mxf, accG, accD

    def post_layer(accG, accD, xp2, ta, td, mxf, b, hh, g, be,
                   wout=None, bout=None):
        resid = [mk_half(_gat_half_body, accG[c], accD[c],
                         xp2[c * N:(c + 1) * N],
                         ta[2 * c:2 * c + 2], td[2 * c:2 * c + 2],
                         mxf[16 * c:16 * c + 16],
                         jnp.broadcast_to(b[None, c * HALF:(c + 1) * HALF],
                                          (N, HALF)),
                         hh[c]) for c in range(NC)]
        st = pl.pallas_call(
            _lnstat_body,
            out_shape=jax.ShapeDtypeStruct((N, 8), F32),
        )(resid[0], resid[1])
        if wout is None:
            return [mk_half(_lnapply_body, resid[c], st,
                            jnp.broadcast_to(g[None, c * HALF:(c + 1) * HALF],
                                             (N, HALF)),
                            jnp.broadcast_to(be[None, c * HALF:(c + 1) * HALF],
                                             (N, HALF))) for c in range(NC)]
        return pl.pallas_call(
            _lnfinal_body,
            out_shape=jax.ShapeDtypeStruct((N,), F32),
        )(resid[0], resid[1], st, g[None, :], be[None, :], wout, bout)

    xp2, ta, td, mxf, accG, accD = gat_layer(h1h, W1, att_src1, att_dst1)
    h2h = post_layer(accG, accD, xp2, ta, td, mxf, b1, h1h, g1, be1)
    xp2b, ta2, td2, mxf2, accG2, accD2 = gat_layer(h2h, W2, att_src2, att_dst2)
    score = post_layer(accG2, accD2, xp2b, ta2, td2, mxf2, b2, h2h, g2, be2,
                       W_out.T, b_out[None, :])
    return score
